# baseline (speedup 1.0000x reference)
import functools

import jax
import jax.numpy as jnp
from jax import lax
from jax.experimental import pallas as pl
from jax.experimental.pallas import tpu as pltpu
from jax.experimental.pallas import tpu_sc as plsc

_NUM_CORES = 2
_NUM_SUBCORES = 16
_NW = _NUM_CORES * _NUM_SUBCORES
_LANES = 128
_BBLK = 512
_NBUF = 2


def _gather_kernel(S, table_hbm, idx_hbm, out_hbm,
                   idx_v0, idx_v1, rows_v0, rows_v1, trans_v0, trans_v1,
                   isem0, isem1, gsem0, gsem1, ssem0, ssem1):
    idx_v = (idx_v0, idx_v1)
    rows_v = (rows_v0, rows_v1)
    trans_v = (trans_v0, trans_v1)
    isem = (isem0, isem1)
    gsem = (gsem0, gsem1)
    ssem = (ssem0, ssem1)
    wid = lax.axis_index("s") * _NUM_CORES + lax.axis_index("c")
    b0 = wid * _BBLK
    tb0 = wid * (_BBLK // _LANES)

    iota = lax.iota(jnp.int32, 16)
    bvecs = [iota + (j * 16) for j in range(_BBLK // 16)]

    def fire_idx(b, g):
        pltpu.async_copy(idx_hbm.at[g, pl.ds(b0, _BBLK)], idx_v[b], isem[b])

    def wait_idx(b):
        pltpu.make_async_copy(idx_hbm.at[0, pl.ds(0, _BBLK)],
                              idx_v[b], isem[b]).wait()

    def fire_gathers(b):
        for j in range(_BBLK // _LANES):
            pltpu.async_copy(
                table_hbm.at[idx_v[b].at[pl.ds(j * _LANES, _LANES)]],
                rows_v[b].at[pl.ds(j * _LANES, _LANES)],
                gsem[b])

    def wait_gathers(b):
        for j in range(_BBLK // _LANES):
            pltpu.make_async_copy(
                table_hbm.at[idx_v[b].at[pl.ds(0, _LANES)]],
                rows_v[b].at[pl.ds(0, _LANES)],
                gsem[b]).wait()

    def fire_store(b, g):
        pltpu.async_copy(trans_v[b], out_hbm.at[g, :, pl.ds(tb0, 4)], ssem[b])

    def wait_store(b):
        pltpu.make_async_copy(trans_v[b],
                              out_hbm.at[0, :, pl.ds(0, 4)], ssem[b]).wait()

    def transpose_all(b):
        @plsc.parallel_loop(0, 32, unroll=8)
        def body(d):
            dvec = jnp.zeros((16,), jnp.int32) + d
            td = lax.div(d, 8)
            sub = lax.rem(d, 8)
            for tbeta in range(4):
                for i in range(8):
                    j = tbeta * 8 + i
                    v = plsc.load_gather(rows_v[b], [bvecs[j], dvec])
                    trans_v[b].at[td, tbeta, sub][pl.ds(i * 16, 16)] = v

    fire_idx(0, 0)
    fire_idx(1, 1)
    wait_idx(0)
    fire_gathers(0)

    def step(g, b, other, first, prefetch, feed):
        wait_gathers(b)
        if prefetch:
            fire_idx(b, g + 2)
        if feed:
            wait_idx(other)
            fire_gathers(other)
        if not first:
            wait_store(b)
        transpose_all(b)
        fire_store(b, g)

    step(0, 0, 1, True, True, True)
    step(1, 1, 0, True, True, True)

    def body(g0, carry):
        for b in range(_NBUF):
            g = g0 * _NBUF + b
            step(g, b, 1 - b, False, True, True)
        return carry

    lax.fori_loop(1, S // _NBUF - 1, body, 0)

    step(S - 2, 0, 1, False, False, True)
    step(S - 1, 1, 0, False, False, False)

    wait_store(0)
    wait_store(1)


def kernel(inputs, embeddings):
    B0, S = inputs.shape
    V, D = embeddings.shape
    assert B0 == _NW * _BBLK and D == 32 and S % _NBUF == 0

    idx_t = inputs.T.astype(jnp.int32)

    mesh = plsc.VectorSubcoreMesh(core_axis_name="c", subcore_axis_name="s")
    k = functools.partial(
        pl.kernel,
        mesh=mesh,
        out_type=jax.ShapeDtypeStruct((S, 4, B0 // _LANES, 8, _LANES),
                                      jnp.float32),
        scratch_types=[
            pltpu.VMEM((_BBLK,), jnp.int32),
            pltpu.VMEM((_BBLK,), jnp.int32),
            pltpu.VMEM((_BBLK, D), jnp.float32),
            pltpu.VMEM((_BBLK, D), jnp.float32),
            pltpu.VMEM((4, 4, 8, _LANES), jnp.float32),
            pltpu.VMEM((4, 4, 8, _LANES), jnp.float32),
            pltpu.SemaphoreType.DMA,
            pltpu.SemaphoreType.DMA,
            pltpu.SemaphoreType.DMA,
            pltpu.SemaphoreType.DMA,
            pltpu.SemaphoreType.DMA,
            pltpu.SemaphoreType.DMA,
        ],
        compiler_params=pltpu.CompilerParams(use_tc_tiling_on_sc=False,
                                             needs_layout_passes=False),
    )(functools.partial(_gather_kernel, S))

    out5 = k(embeddings, idx_t)
    out = out5.transpose(2, 4, 0, 1, 3).reshape(B0, S, D)
    return out


# speedup vs baseline: 1.1466x; 1.1466x over previous
import functools

import jax
import jax.numpy as jnp
from jax import lax
from jax.experimental import pallas as pl
from jax.experimental.pallas import tpu as pltpu
from jax.experimental.pallas import tpu_sc as plsc

_NUM_CORES = 2
_NUM_SUBCORES = 16
_NW = _NUM_CORES * _NUM_SUBCORES
_LANES = 128
_BBLK = 512
_NBUF = 2


def _gather_kernel(S, table_hbm, idx_hbm, out_hbm,
                   idx_v0, idx_v1, rows_v0, rows_v1, trans_v0, trans_v1,
                   isem0, isem1, gsem0, gsem1, ssem0, ssem1):
    idx_v = (idx_v0, idx_v1)
    rows_v = (rows_v0, rows_v1)
    trans_v = (trans_v0, trans_v1)
    isem = (isem0, isem1)
    gsem = (gsem0, gsem1)
    ssem = (ssem0, ssem1)
    wid = lax.axis_index("s") * _NUM_CORES + lax.axis_index("c")
    b0 = wid * _BBLK
    tb0 = wid * (_BBLK // _LANES)

    iota = lax.iota(jnp.int32, 16)
    bvecs = [iota + (j * 16) for j in range(_BBLK // 16)]

    def fire_idx(b, g):
        pltpu.async_copy(idx_hbm.at[g, pl.ds(b0, _BBLK)], idx_v[b], isem[b])

    def wait_idx(b):
        pltpu.make_async_copy(idx_hbm.at[0, pl.ds(0, _BBLK)],
                              idx_v[b], isem[b]).wait()

    def fire_gathers(b):
        for j in range(_BBLK // _LANES):
            pltpu.async_copy(
                table_hbm.at[idx_v[b].at[pl.ds(j * _LANES, _LANES)]],
                rows_v[b].at[pl.ds(j * _LANES, _LANES)],
                gsem[b])

    def wait_gathers(b):
        for j in range(_BBLK // _LANES):
            pltpu.make_async_copy(
                table_hbm.at[idx_v[b].at[pl.ds(0, _LANES)]],
                rows_v[b].at[pl.ds(0, _LANES)],
                gsem[b]).wait()

    def fire_store(b, g):
        pltpu.async_copy(trans_v[b], out_hbm.at[g, :, pl.ds(tb0, 4)], ssem[b])

    def wait_store(b):
        pltpu.make_async_copy(trans_v[b],
                              out_hbm.at[0, :, pl.ds(0, 4)], ssem[b]).wait()

    def transpose_all(b):
        @plsc.parallel_loop(0, 32, unroll=4)
        def body(d):
            dvec = jnp.zeros((16,), jnp.int32) + d
            td = lax.div(d, 8)
            sub = lax.rem(d, 8)
            for tbeta in range(4):
                for i in range(8):
                    j = tbeta * 8 + i
                    v = plsc.load_gather(rows_v[b], [bvecs[j], dvec])
                    trans_v[b].at[td, tbeta, sub][pl.ds(i * 16, 16)] = v

    fire_idx(0, 0)
    fire_idx(1, 1)
    wait_idx(0)
    fire_gathers(0)

    def step(g, b, other, first, prefetch, feed):
        wait_gathers(b)
        if prefetch:
            fire_idx(b, g + 2)
        if feed:
            wait_idx(other)
            fire_gathers(other)
        if not first:
            wait_store(b)
        transpose_all(b)
        fire_store(b, g)

    step(0, 0, 1, True, True, True)
    step(1, 1, 0, True, True, True)

    def body(g0, carry):
        for b in range(_NBUF):
            g = g0 * _NBUF + b
            step(g, b, 1 - b, False, True, True)
        return carry

    lax.fori_loop(1, S // _NBUF - 1, body, 0)

    step(S - 2, 0, 1, False, False, True)
    step(S - 1, 1, 0, False, False, False)

    wait_store(0)
    wait_store(1)


def kernel(inputs, embeddings):
    B0, S = inputs.shape
    V, D = embeddings.shape
    assert B0 == _NW * _BBLK and D == 32 and S % _NBUF == 0

    idx_t = inputs.T.astype(jnp.int32)

    mesh = plsc.VectorSubcoreMesh(core_axis_name="c", subcore_axis_name="s")
    k = functools.partial(
        pl.kernel,
        mesh=mesh,
        out_type=jax.ShapeDtypeStruct((S, 4, B0 // _LANES, 8, _LANES),
                                      jnp.float32),
        scratch_types=[
            pltpu.VMEM((_BBLK,), jnp.int32),
            pltpu.VMEM((_BBLK,), jnp.int32),
            pltpu.VMEM((_BBLK, D), jnp.float32),
            pltpu.VMEM((_BBLK, D), jnp.float32),
            pltpu.VMEM((4, 4, 8, _LANES), jnp.float32),
            pltpu.VMEM((4, 4, 8, _LANES), jnp.float32),
            pltpu.SemaphoreType.DMA,
            pltpu.SemaphoreType.DMA,
            pltpu.SemaphoreType.DMA,
            pltpu.SemaphoreType.DMA,
            pltpu.SemaphoreType.DMA,
            pltpu.SemaphoreType.DMA,
        ],
        compiler_params=pltpu.CompilerParams(use_tc_tiling_on_sc=False,
                                             needs_layout_passes=False),
    )(functools.partial(_gather_kernel, S))

    out5 = k(embeddings, idx_t)
    out = out5.transpose(2, 4, 0, 1, 3).reshape(B0, S, D)
    return out
